# baseline (speedup 1.0000x reference)
import functools

import jax
import jax.numpy as jnp
from jax import lax
from jax.experimental import pallas as pl
from jax.experimental.pallas import tpu as pltpu
from jax.experimental.pallas import tpu_sc as plsc

DIM = 128
HDIM = DIM // 2
B = 16384
NUM_CARDS = 7
NCHUNK = DIM // 16
NPAIR = NCHUNK // 2

_info = plsc.get_sparse_core_info()
NC, NS = _info.num_cores, _info.num_subcores
NW = NC * NS
ROWS_PER_W = B // NW
GROUPS = ROWS_PER_W // 16


def _sc_body(inp_hbm, card_hbm, rank_hbm, suit_hbm, out_hbm,
             inp_v, row_v, card_v, rank_v, suit_v, comb_v, out_v, out_sem):
  wid = lax.axis_index("s") * NC + lax.axis_index("c")
  row0 = wid * ROWS_PER_W

  pltpu.sync_copy(card_hbm, card_v)
  pltpu.sync_copy(rank_hbm, rank_v)
  pltpu.sync_copy(suit_hbm, suit_v)
  for j in range(NUM_CARDS):
    pltpu.sync_copy(inp_hbm.at[pl.ds(j, 1), pl.ds(row0, ROWS_PER_W)],
                    inp_v.at[pl.ds(j, 1), :])

  @pl.loop(0, 52)
  def _build(c):
    r = lax.shift_right_logical(c, 2)
    s = lax.bitwise_and(c, 3)
    cb = c * DIM
    rb = r * DIM
    sb = s * DIM
    ch = []
    for k in range(NCHUNK):
      ch.append(card_v[pl.ds(cb + k * 16, 16)]
                + rank_v[pl.ds(rb + k * 16, 16)]
                + suit_v[pl.ds(sb + k * 16, 16)])
    hb = c * HDIM
    for p in range(NPAIR):
      packed = plsc.pack(ch[2 * p], ch[2 * p + 1],
                         format=plsc.PackFormat.INTERLEAVED)
      comb_v[pl.ds(hb + p * 16, 16)] = plsc.bitcast(packed, jnp.int32)

  iota16 = lax.iota(jnp.int32, 16)
  idx0 = iota16 * NUM_CARDS

  @plsc.parallel_loop(0, GROUPS)
  def _reorder(g):
    for j in range(NUM_CARDS):
      v = inp_v[j, pl.ds(g * 16, 16)] * HDIM
      plsc.store_scatter(row_v, [idx0 + (g * 16 * NUM_CARDS + j)], v)

  copies = []
  qrows = ROWS_PER_W // 4
  for q in range(4):

    @plsc.parallel_loop(q * qrows, (q + 1) * qrows, unroll=2)
    def _row(b):
      iv = row_v[pl.ds(b * NUM_CARDS, 16)]
      bases = [iv[j] for j in range(NUM_CARDS)]
      for p in range(NPAIR):
        g7 = [plsc.bitcast(comb_v[pl.ds(bases[j] + p * 16, 16)],
                           jnp.bfloat16) for j in range(NUM_CARDS)]
        t01 = g7[0] + g7[1]
        t23 = g7[2] + g7[3]
        t45 = g7[4] + g7[5]
        acc = (t01 + t23) + (t45 + g7[6])
        lo, hi = plsc.unpack(acc, format=plsc.PackFormat.INTERLEAVED)
        out_v[b, pl.ds(p * 32, 16)] = lo
        out_v[b, pl.ds(p * 32 + 16, 16)] = hi

    copies.append(pltpu.async_copy(
        out_v.at[pl.ds(q * qrows, qrows), :],
        out_hbm.at[pl.ds(row0 + q * qrows, qrows), :], out_sem))
  for cp in copies:
    cp.wait()


@jax.jit
def _card_embed(inp_t, card_flat, rank_flat, suit_flat):
  mesh = plsc.VectorSubcoreMesh(core_axis_name="c", subcore_axis_name="s")
  kern = pl.kernel(
      _sc_body,
      out_type=jax.ShapeDtypeStruct((B, DIM), jnp.float32),
      mesh=mesh,
      compiler_params=pltpu.CompilerParams(needs_layout_passes=False),
      scratch_types=[
          pltpu.VMEM((NUM_CARDS, ROWS_PER_W), jnp.int32),
          pltpu.VMEM((ROWS_PER_W * NUM_CARDS + 16,), jnp.int32),
          pltpu.VMEM((52 * DIM,), jnp.float32),
          pltpu.VMEM((13 * DIM,), jnp.float32),
          pltpu.VMEM((4 * DIM,), jnp.float32),
          pltpu.VMEM((52 * HDIM,), jnp.int32),
          pltpu.VMEM((ROWS_PER_W, DIM), jnp.float32),
          pltpu.SemaphoreType.DMA,
      ],
  )
  return kern(inp_t, card_flat, rank_flat, suit_flat)


def kernel(input, rank_table, suit_table, card_table):
  inp_t = input.astype(jnp.int32).T
  return _card_embed(inp_t, card_table.reshape(-1),
                     rank_table.reshape(-1), suit_table.reshape(-1))


# speedup vs baseline: 1.1256x; 1.1256x over previous
import functools

import jax
import jax.numpy as jnp
from jax import lax
from jax.experimental import pallas as pl
from jax.experimental.pallas import tpu as pltpu
from jax.experimental.pallas import tpu_sc as plsc

DIM = 128
HDIM = DIM // 2
B = 16384
NUM_CARDS = 7
NCHUNK = DIM // 16
NPAIR = NCHUNK // 2

_info = plsc.get_sparse_core_info()
NC, NS = _info.num_cores, _info.num_subcores
NW = NC * NS
ROWS_PER_W = B // NW
GROUPS = ROWS_PER_W // 16


def _sc_body(inp_hbm, card_hbm, rank_hbm, suit_hbm, out_hbm,
             inp_v, row_v, card_v, rank_v, suit_v, comb_v, out_v, out_sem):
  wid = lax.axis_index("s") * NC + lax.axis_index("c")
  row0 = wid * ROWS_PER_W

  stage = [
      pltpu.async_copy(card_hbm, card_v, out_sem),
      pltpu.async_copy(rank_hbm, rank_v, out_sem),
      pltpu.async_copy(suit_hbm, suit_v, out_sem),
  ]
  for j in range(NUM_CARDS):
    stage.append(
        pltpu.async_copy(inp_hbm.at[pl.ds(j, 1), pl.ds(row0, ROWS_PER_W)],
                         inp_v.at[pl.ds(j, 1), :], out_sem))
  for cp in stage:
    cp.wait()

  @pl.loop(0, 52)
  def _build(c):
    r = lax.shift_right_logical(c, 2)
    s = lax.bitwise_and(c, 3)
    cb = c * DIM
    rb = r * DIM
    sb = s * DIM
    ch = []
    for k in range(NCHUNK):
      ch.append(card_v[pl.ds(cb + k * 16, 16)]
                + rank_v[pl.ds(rb + k * 16, 16)]
                + suit_v[pl.ds(sb + k * 16, 16)])
    hb = c * HDIM
    for p in range(NPAIR):
      packed = plsc.pack(ch[2 * p], ch[2 * p + 1],
                         format=plsc.PackFormat.INTERLEAVED)
      comb_v[pl.ds(hb + p * 16, 16)] = plsc.bitcast(packed, jnp.int32)

  iota16 = lax.iota(jnp.int32, 16)
  idx0 = iota16 * NUM_CARDS

  @plsc.parallel_loop(0, GROUPS)
  def _reorder(g):
    for j in range(NUM_CARDS):
      v = inp_v[j, pl.ds(g * 16, 16)] * HDIM
      plsc.store_scatter(row_v, [idx0 + (g * 16 * NUM_CARDS + j)], v)

  @plsc.parallel_loop(0, ROWS_PER_W, unroll=2)
  def _row(b):
    iv = row_v[pl.ds(b * NUM_CARDS, 16)]
    bases = [iv[j] for j in range(NUM_CARDS)]
    for p in range(NPAIR):
      g7 = [plsc.bitcast(comb_v[pl.ds(bases[j] + p * 16, 16)],
                         jnp.bfloat16) for j in range(NUM_CARDS)]
      t01 = g7[0] + g7[1]
      t23 = g7[2] + g7[3]
      t45 = g7[4] + g7[5]
      acc = (t01 + t23) + (t45 + g7[6])
      lo, hi = plsc.unpack(acc, format=plsc.PackFormat.INTERLEAVED)
      out_v[b, pl.ds(p * 32, 16)] = lo
      out_v[b, pl.ds(p * 32 + 16, 16)] = hi

  pltpu.sync_copy(out_v, out_hbm.at[pl.ds(row0, ROWS_PER_W), :])


@jax.jit
def _card_embed(inp_t, card_flat, rank_flat, suit_flat):
  mesh = plsc.VectorSubcoreMesh(core_axis_name="c", subcore_axis_name="s")
  kern = pl.kernel(
      _sc_body,
      out_type=jax.ShapeDtypeStruct((B, DIM), jnp.float32),
      mesh=mesh,
      compiler_params=pltpu.CompilerParams(needs_layout_passes=False),
      scratch_types=[
          pltpu.VMEM((NUM_CARDS, ROWS_PER_W), jnp.int32),
          pltpu.VMEM((ROWS_PER_W * NUM_CARDS + 16,), jnp.int32),
          pltpu.VMEM((52 * DIM,), jnp.float32),
          pltpu.VMEM((13 * DIM,), jnp.float32),
          pltpu.VMEM((4 * DIM,), jnp.float32),
          pltpu.VMEM((52 * HDIM,), jnp.int32),
          pltpu.VMEM((ROWS_PER_W, DIM), jnp.float32),
          pltpu.SemaphoreType.DMA,
      ],
  )
  return kern(inp_t, card_flat, rank_flat, suit_flat)


def kernel(input, rank_table, suit_table, card_table):
  inp_t = input.astype(jnp.int32).T
  return _card_embed(inp_t, card_table.reshape(-1),
                     rank_table.reshape(-1), suit_table.reshape(-1))


# speedup vs baseline: 1.1440x; 1.0164x over previous
import functools

import jax
import jax.numpy as jnp
from jax import lax
from jax.experimental import pallas as pl
from jax.experimental.pallas import tpu as pltpu
from jax.experimental.pallas import tpu_sc as plsc

DIM = 128
HDIM = DIM // 2
B = 16384
NUM_CARDS = 7
NCHUNK = DIM // 16
NPAIR = NCHUNK // 2

_info = plsc.get_sparse_core_info()
NC, NS = _info.num_cores, _info.num_subcores
NW = NC * NS
ROWS_PER_W = B // NW
GROUPS = ROWS_PER_W // 16


def _sc_body(inp_hbm, card_hbm, rank_hbm, suit_hbm, out_hbm,
             inp_v, row_v, card_v, rank_v, suit_v, comb_v, out_v, out_sem):
  wid = lax.axis_index("s") * NC + lax.axis_index("c")
  row0 = wid * ROWS_PER_W

  stage = [
      pltpu.async_copy(card_hbm, card_v, out_sem),
      pltpu.async_copy(rank_hbm, rank_v, out_sem),
      pltpu.async_copy(suit_hbm, suit_v, out_sem),
  ]
  inp_cps = [
      pltpu.async_copy(inp_hbm.at[pl.ds(j, 1), pl.ds(row0, ROWS_PER_W)],
                       inp_v.at[pl.ds(j, 1), :], out_sem)
      for j in range(NUM_CARDS)
  ]
  for cp in stage:
    cp.wait()

  @pl.loop(0, 52)
  def _build(c):
    r = lax.shift_right_logical(c, 2)
    s = lax.bitwise_and(c, 3)
    cb = c * DIM
    rb = r * DIM
    sb = s * DIM
    ch = []
    for k in range(NCHUNK):
      ch.append(card_v[pl.ds(cb + k * 16, 16)]
                + rank_v[pl.ds(rb + k * 16, 16)]
                + suit_v[pl.ds(sb + k * 16, 16)])
    hb = c * HDIM
    for p in range(NPAIR):
      packed = plsc.pack(ch[2 * p], ch[2 * p + 1],
                         format=plsc.PackFormat.INTERLEAVED)
      comb_v[pl.ds(hb + p * 16, 16)] = plsc.bitcast(packed, jnp.int32)

  for cp in inp_cps:
    cp.wait()
  iota16 = lax.iota(jnp.int32, 16)
  idx0 = iota16 * NUM_CARDS

  @plsc.parallel_loop(0, GROUPS)
  def _reorder(g):
    for j in range(NUM_CARDS):
      v = inp_v[j, pl.ds(g * 16, 16)] * HDIM
      plsc.store_scatter(row_v, [idx0 + (g * 16 * NUM_CARDS + j)], v)

  half = ROWS_PER_W // 2
  copies = []
  for q in range(2):

    @plsc.parallel_loop(q * half, (q + 1) * half, unroll=2)
    def _row(b):
      iv = row_v[pl.ds(b * NUM_CARDS, 16)]
      bases = [iv[j] for j in range(NUM_CARDS)]
      for p in range(NPAIR):
        g7 = [plsc.bitcast(comb_v[pl.ds(bases[j] + p * 16, 16)],
                           jnp.bfloat16) for j in range(NUM_CARDS)]
        t01 = g7[0] + g7[1]
        t23 = g7[2] + g7[3]
        t45 = g7[4] + g7[5]
        acc = (t01 + t23) + (t45 + g7[6])
        lo, hi = plsc.unpack(acc, format=plsc.PackFormat.INTERLEAVED)
        out_v[b, pl.ds(p * 32, 16)] = lo
        out_v[b, pl.ds(p * 32 + 16, 16)] = hi

    copies.append(pltpu.async_copy(
        out_v.at[pl.ds(q * half, half), :],
        out_hbm.at[pl.ds(row0 + q * half, half), :], out_sem))
  for cp in copies:
    cp.wait()


@jax.jit
def _card_embed(inp_t, card_flat, rank_flat, suit_flat):
  mesh = plsc.VectorSubcoreMesh(core_axis_name="c", subcore_axis_name="s")
  kern = pl.kernel(
      _sc_body,
      out_type=jax.ShapeDtypeStruct((B, DIM), jnp.float32),
      mesh=mesh,
      compiler_params=pltpu.CompilerParams(needs_layout_passes=False),
      scratch_types=[
          pltpu.VMEM((NUM_CARDS, ROWS_PER_W), jnp.int32),
          pltpu.VMEM((ROWS_PER_W * NUM_CARDS + 16,), jnp.int32),
          pltpu.VMEM((52 * DIM,), jnp.float32),
          pltpu.VMEM((13 * DIM,), jnp.float32),
          pltpu.VMEM((4 * DIM,), jnp.float32),
          pltpu.VMEM((52 * HDIM,), jnp.int32),
          pltpu.VMEM((ROWS_PER_W, DIM), jnp.float32),
          pltpu.SemaphoreType.DMA,
      ],
  )
  return kern(inp_t, card_flat, rank_flat, suit_flat)


def kernel(input, rank_table, suit_table, card_table):
  inp_t = input.astype(jnp.int32).T
  return _card_embed(inp_t, card_table.reshape(-1),
                     rank_table.reshape(-1), suit_table.reshape(-1))
